# baseline (speedup 1.0000x reference)
import functools

import jax
import jax.numpy as jnp
from jax import lax
from jax.experimental import pallas as pl
from jax.experimental.pallas import tpu as pltpu
from jax.experimental.pallas import tpu_sc as plsc

D = 32
DW = 128
N_NEG = 100
ITEM_ROWS = 1000000
NC = 2
NS = 16
NW = NC * NS
CB = 8
CROWS = CB * N_NEG


def _tower_body(bpw, uid_hbm, pid_hbm, nid_hbm, ucol_hbm, pcol_hbm, ncol_hbm,
                utab_hbm, itab_hbm,
                pos_out_hbm, neg_out_hbm,
                uid_v, pid_v, nid_v, ucol_v, pcol_v, ncol_v,
                urows_v, prows_v, nrows_v,
                posres_v, negres_v, sem):
    wid = lax.axis_index("s") * NC + lax.axis_index("c")
    nchunk = bpw // CB

    def chunk_body(c, _):
        b0 = wid * bpw + c * CB
        pltpu.sync_copy(uid_hbm.at[pl.ds(b0, CB)], uid_v.at[pl.ds(0, CB)])
        pltpu.sync_copy(pid_hbm.at[pl.ds(b0, CB)], pid_v.at[pl.ds(0, CB)])
        pltpu.sync_copy(nid_hbm.at[pl.ds(b0 * N_NEG, CROWS)], nid_v)
        pltpu.sync_copy(ucol_hbm.at[pl.ds(b0, CB)], ucol_v.at[pl.ds(0, CB)])
        pltpu.sync_copy(pcol_hbm.at[pl.ds(b0, CB)], pcol_v.at[pl.ds(0, CB)])
        pltpu.sync_copy(ncol_hbm.at[pl.ds(b0 * N_NEG, CROWS)], ncol_v.at[pl.ds(0, CROWS)])
        cu = pltpu.async_copy(utab_hbm.at[uid_v.at[pl.ds(0, CB)]],
                              urows_v, sem)
        cp = pltpu.async_copy(itab_hbm.at[pid_v.at[pl.ds(0, CB)]],
                              prows_v, sem)
        cn = pltpu.async_copy(itab_hbm.at[nid_v], nrows_v, sem)
        cu.wait()
        cp.wait()
        cn.wait()

        lane = lax.iota(jnp.int32, 16)

        acc = jnp.zeros(16, jnp.float32)
        for j in range(CB):
            uc = ucol_v[pl.ds(j, 16)][0]
            pc = pcol_v[pl.ds(j, 16)][0]
            q0 = urows_v[j, pl.ds(uc, 16)]
            q1 = urows_v[j, pl.ds(uc + 16, 16)]
            p0 = prows_v[j, pl.ds(pc, 16)]
            p1 = prows_v[j, pl.ds(pc + 16, 16)]
            acc = jnp.where(lane == j, jnp.sum(p0 * q0 + p1 * q1), acc)
        posres_v[...] = acc
        pltpu.sync_copy(posres_v.at[pl.ds(0, CB)],
                        pos_out_hbm.at[pl.ds(b0, CB)])

        def b_body(i, _):
            uc = ucol_v[pl.ds(i, 16)][0]
            q0 = urows_v[i, pl.ds(uc, 16)]
            q1 = urows_v[i, pl.ds(uc + 16, 16)]
            r_base = i * N_NEG
            for n0 in (0, 16, 32, 48, 64, 80, 84):
                acc = jnp.zeros(16, jnp.float32)
                for j in range(16):
                    r = r_base + n0 + j
                    nc = ncol_v[pl.ds(r, 16)][0]
                    e0 = nrows_v[r, pl.ds(nc, 16)]
                    e1 = nrows_v[r, pl.ds(nc + 16, 16)]
                    acc = jnp.where(lane == j, jnp.sum(e0 * q0 + e1 * q1), acc)
                plsc.store_scatter(negres_v, [r_base + n0 + lane], acc)
            return 0

        lax.fori_loop(0, CB, b_body, 0)
        pltpu.sync_copy(negres_v, neg_out_hbm.at[pl.ds(b0 * N_NEG, CROWS)])
        return 0

    lax.fori_loop(0, nchunk, chunk_body, 0)


def kernel(user_id, pos_items, neg_items, user_table, item_table):
    b = user_id.shape[0]
    bpw = b // NW
    neg_flat = neg_items.reshape(-1)
    mesh = plsc.VectorSubcoreMesh(core_axis_name="c", subcore_axis_name="s")
    run = pl.kernel(
        functools.partial(_tower_body, bpw),
        out_type=(
            jax.ShapeDtypeStruct((b,), jnp.float32),
            jax.ShapeDtypeStruct((b * N_NEG,), jnp.float32),
        ),
        mesh=mesh,
        compiler_params=pltpu.CompilerParams(needs_layout_passes=False),
        scratch_types=[
            pltpu.VMEM((16,), jnp.int32),
            pltpu.VMEM((16,), jnp.int32),
            pltpu.VMEM((CROWS,), jnp.int32),
            pltpu.VMEM((32,), jnp.int32),
            pltpu.VMEM((32,), jnp.int32),
            pltpu.VMEM((CROWS + 16,), jnp.int32),
            pltpu.VMEM((CB, DW), jnp.float32),
            pltpu.VMEM((CB, DW), jnp.float32),
            pltpu.VMEM((CROWS, DW), jnp.float32),
            pltpu.VMEM((16,), jnp.float32),
            pltpu.VMEM((CROWS,), jnp.float32),
            pltpu.SemaphoreType.DMA,
        ],
    )
    pos_score, neg_score_flat = run(
        user_id >> 2, pos_items >> 2, neg_flat >> 2,
        (user_id & 3) * D, (pos_items & 3) * D, (neg_flat & 3) * D,
        user_table.reshape(-1, DW),
        item_table[:ITEM_ROWS].reshape(-1, DW))
    return pos_score, neg_score_flat.reshape(b, N_NEG)


# speedup vs baseline: 1.5012x; 1.5012x over previous
import functools

import jax
import jax.numpy as jnp
from jax import lax
from jax.experimental import pallas as pl
from jax.experimental.pallas import tpu as pltpu
from jax.experimental.pallas import tpu_sc as plsc

D = 32
N_NEG = 100
NC = 2
NS = 16
NW = NC * NS
CB = 16
CROWS = CB * N_NEG


def _tower_body(bpw, uid_hbm, pid_hbm, nid_hbm, utab_hbm, itab_hbm,
                pos_out_hbm, neg_out_hbm,
                uid_a, pid_a, nid_a, urows_a, prows_a, nrows_a, sem_a,
                uid_b, pid_b, nid_b, urows_b, prows_b, nrows_b, sem_b,
                posres_v, negres_v):
    wid = lax.axis_index("s") * NC + lax.axis_index("c")
    nchunk = bpw // CB
    bufs = (
        (uid_a, pid_a, nid_a, urows_a, prows_a, nrows_a, sem_a),
        (uid_b, pid_b, nid_b, urows_b, prows_b, nrows_b, sem_b),
    )

    def issue(ch, buf):
        uid_v, pid_v, nid_v, urows_v, prows_v, nrows_v, sem = buf
        b0 = wid * bpw + ch * CB
        pltpu.sync_copy(uid_hbm.at[pl.ds(b0, CB)], uid_v)
        pltpu.sync_copy(pid_hbm.at[pl.ds(b0, CB)], pid_v)
        pltpu.sync_copy(nid_hbm.at[pl.ds(b0 * N_NEG, CROWS)], nid_v)
        pltpu.async_copy(utab_hbm.at[uid_v], urows_v, sem)
        pltpu.async_copy(itab_hbm.at[pid_v], prows_v, sem)
        pltpu.async_copy(itab_hbm.at[nid_v], nrows_v, sem)

    def wait_gathers(buf):
        uid_v, pid_v, nid_v, urows_v, prows_v, nrows_v, sem = buf
        pltpu.make_async_copy(utab_hbm.at[uid_v], urows_v, sem).wait()
        pltpu.make_async_copy(itab_hbm.at[pid_v], prows_v, sem).wait()
        pltpu.make_async_copy(itab_hbm.at[nid_v], nrows_v, sem).wait()

    def compute(ch, buf):
        uid_v, pid_v, nid_v, urows_v, prows_v, nrows_v, sem = buf
        b0 = wid * bpw + ch * CB
        lane = lax.iota(jnp.int32, 16)

        acc = jnp.zeros(16, jnp.float32)
        for j in range(CB):
            q0 = urows_v[j, pl.ds(0, 16)]
            q1 = urows_v[j, pl.ds(16, 16)]
            p0 = prows_v[j, pl.ds(0, 16)]
            p1 = prows_v[j, pl.ds(16, 16)]
            acc = jnp.where(lane == j, jnp.sum(p0 * q0 + p1 * q1), acc)
        posres_v[...] = acc
        pltpu.sync_copy(posres_v, pos_out_hbm.at[pl.ds(b0, CB)])

        def b_body(i, _):
            q0 = urows_v[i, pl.ds(0, 16)]
            q1 = urows_v[i, pl.ds(16, 16)]
            r_base = i * N_NEG
            for n0 in (0, 16, 32, 48, 64, 80, 84):
                acc = jnp.zeros(16, jnp.float32)
                for j in range(16):
                    r = r_base + n0 + j
                    e0 = nrows_v[r, pl.ds(0, 16)]
                    e1 = nrows_v[r, pl.ds(16, 16)]
                    acc = jnp.where(lane == j, jnp.sum(e0 * q0 + e1 * q1), acc)
                plsc.store_scatter(negres_v, [r_base + n0 + lane], acc)
            return 0

        lax.fori_loop(0, CB, b_body, 0)
        pltpu.sync_copy(negres_v, neg_out_hbm.at[pl.ds(b0 * N_NEG, CROWS)])

    issue(0, bufs[0])

    def pair_body(c, _):
        ch = 2 * c
        wait_gathers(bufs[0])
        issue(ch + 1, bufs[1])
        compute(ch, bufs[0])
        wait_gathers(bufs[1])
        issue(ch + 2, bufs[0])
        compute(ch + 1, bufs[1])
        return 0

    lax.fori_loop(0, nchunk // 2 - 1, pair_body, 0)

    wait_gathers(bufs[0])
    issue(nchunk - 1, bufs[1])
    compute(nchunk - 2, bufs[0])
    wait_gathers(bufs[1])
    compute(nchunk - 1, bufs[1])


def kernel(user_id, pos_items, neg_items, user_table, item_table):
    b = user_id.shape[0]
    bpw = b // NW
    neg_flat = neg_items.reshape(-1)
    mesh = plsc.VectorSubcoreMesh(core_axis_name="c", subcore_axis_name="s")
    dbuf = [
        pltpu.VMEM((CB,), jnp.int32),
        pltpu.VMEM((CB,), jnp.int32),
        pltpu.VMEM((CROWS,), jnp.int32),
        pltpu.VMEM((CB, D), jnp.float32),
        pltpu.VMEM((CB, D), jnp.float32),
        pltpu.VMEM((CROWS, D), jnp.float32),
        pltpu.SemaphoreType.DMA,
    ]
    run = pl.kernel(
        functools.partial(_tower_body, bpw),
        out_type=(
            jax.ShapeDtypeStruct((b,), jnp.float32),
            jax.ShapeDtypeStruct((b * N_NEG,), jnp.float32),
        ),
        mesh=mesh,
        compiler_params=pltpu.CompilerParams(
            needs_layout_passes=False, use_tc_tiling_on_sc=False),
        scratch_types=dbuf + dbuf + [
            pltpu.VMEM((16,), jnp.float32),
            pltpu.VMEM((CROWS,), jnp.float32),
        ],
    )
    pos_score, neg_score_flat = run(user_id, pos_items, neg_flat,
                                    user_table, item_table)
    return pos_score, neg_score_flat.reshape(b, N_NEG)
